# baseline (speedup 1.0000x reference)
import functools

import jax
import jax.numpy as jnp
from jax import lax
from jax.experimental import pallas as pl
from jax.experimental.pallas import tpu as pltpu
from jax.experimental.pallas import tpu_sc as plsc

N_NODES = 100000
N_EDGES = 3200000

W_VCR = 1.0
W_FLOW = 0.005
W_CONS = 0.05

TGT_SCALE, TGT_SHIFT = 0.8, 0.5
CAP_SCALE, CAP_SHIFT = 1500.0, 2000.0
FLOW_SCALE, FLOW_SHIFT = 1200.0, 0.0
DEM_SCALE, DEM_SHIFT = 500.0, 0.0

NW = 32
LANE = 128
CR = 40
CE = CR * LANE
NCHUNKS = N_EDGES // CE
CHUNK_LO = NCHUNKS // NW
CHUNK_EXTRA = NCHUNKS - CHUNK_LO * NW
NBLK = CHUNK_LO + 1
NODES_PAD = 100096
SLICE = NODES_PAD // 16
NSETS = 3

G0 = 25
BE = N_EDGES // G0


def _k0_body(p_ref, y_ref, c_ref, f_ref, pf_ref, part_ref):
    p = p_ref[...]
    yy = y_ref[...]
    cc = c_ref[...]
    ff = f_ref[...]
    pf = (p * TGT_SCALE + TGT_SHIFT) * (cc * CAP_SCALE + CAP_SHIFT)
    pf_ref[...] = pf
    d1 = p - yy
    d2 = pf - (ff * FLOW_SCALE + FLOW_SHIFT)
    part_ref[0, 0, :] = jnp.sum((d1 * d1).reshape(-1, LANE), axis=0)
    part_ref[0, 1, :] = jnp.sum((d2 * d2).reshape(-1, LANE), axis=0)


_k0 = pl.pallas_call(
    _k0_body,
    grid=(G0,),
    in_specs=[pl.BlockSpec((BE,), lambda i: (i,))] * 4,
    out_specs=[
        pl.BlockSpec((BE,), lambda i: (i,)),
        pl.BlockSpec((1, 2, LANE), lambda i: (i, 0, 0)),
    ],
    out_shape=[
        jax.ShapeDtypeStruct((N_EDGES,), jnp.float32),
        jax.ShapeDtypeStruct((G0, 2, LANE), jnp.float32),
    ],
)


def _k1_body(pf_h, eix_h, acc_o, *scr):
    cid = lax.axis_index("c")
    sid = lax.axis_index("s")
    w = cid * 16 + sid
    acc_sh = scr[5 * NSETS]
    isems = scr[5 * NSETS + 1:5 * NSETS + 1 + NSETS]
    ssems = scr[5 * NSETS + 1 + NSETS:]
    bufs = [tuple(scr[5 * s:5 * s + 5]) + (isems[s], ssems[s])
            for s in range(NSETS)]
    pfv0 = bufs[0][1]

    def zstep(j, carry):
        for u in range(LANE // 16):
            pfv0[pl.ds(j * LANE + u * 16, 16)] = jnp.zeros((16,), jnp.float32)
        return carry

    lax.fori_loop(0, CR, zstep, 0)
    pltpu.sync_copy(pfv0, acc_sh.at[pl.ds(sid * SLICE, CE)])
    pltpu.sync_copy(pfv0.at[pl.ds(0, SLICE - CE)],
                    acc_sh.at[pl.ds(sid * SLICE + CE, SLICE - CE)])
    plsc.subcore_barrier()

    wmin = jnp.minimum(w, CHUNK_EXTRA)
    base_chunk = CHUNK_LO * w + wmin
    n_chunks = CHUNK_LO + jnp.where(w < CHUNK_EXTRA, 1, 0)

    def prefetch(g, s):
        pin, _, _, sv, dv, isem, _ = bufs[s]
        e0 = g * CE
        pltpu.async_copy(pf_h.at[pl.ds(e0, CE)], pin, isem)
        pltpu.async_copy(eix_h.at[pl.ds(e0, CE)], sv, isem)
        pltpu.async_copy(eix_h.at[pl.ds(N_EDGES + e0, CE)], dv, isem)

    def wait_in(s):
        pin, _, _, sv, dv, isem, _ = bufs[s]
        for dst in (pin, sv, dv):
            pltpu.make_async_copy(pf_h.at[pl.ds(0, CE)], dst, isem).wait()

    def compute(s, mvec):
        pin, pfv, npfv, _, _, _, _ = bufs[s]

        def estep(j, carry):
            for u in range(LANE // 16):
                o = j * LANE + u * 16
                x = pin[pl.ds(o, 16)]
                if mvec is not None:
                    x = x * mvec
                pfv[pl.ds(o, 16)] = x
                npfv[pl.ds(o, 16)] = -x
            return carry

        lax.fori_loop(0, CR, estep, 0)

    def fire(s):
        _, pfv, npfv, sv, dv, _, ssem = bufs[s]
        pltpu.async_copy(pfv, acc_sh.at[dv], ssem, add=True)
        pltpu.async_copy(npfv, acc_sh.at[sv], ssem, add=True)

    def drain(s):
        _, pfv, npfv, sv, dv, _, ssem = bufs[s]
        pltpu.make_async_copy(pfv, acc_sh.at[dv], ssem).wait()
        pltpu.make_async_copy(npfv, acc_sh.at[sv], ssem).wait()

    for s in range(NSETS):
        prefetch(base_chunk + s, s)

    wait_in(0)
    compute(0, None)
    fire(0)
    wait_in(1)
    compute(1, None)
    fire(1)

    def body(k, carry):
        for off in (2, 3, 4):
            s = off % NSETS
            c = 3 * k + off
            wait_in(s)
            sd = (s + 1) % NSETS
            drain(sd)
            prefetch(jnp.minimum(base_chunk + c + 1, NCHUNKS - 1), sd)
            if off == 4:
                mvec = jnp.zeros((16,), jnp.float32) + jnp.where(
                    c < n_chunks, 1.0, 0.0).astype(jnp.float32)
            else:
                mvec = None
            compute(s, mvec)
            fire(s)
        return carry

    lax.fori_loop(0, (NBLK - 2) // 3, body, 0)
    drain(0)
    drain(1)
    wait_in(2)

    plsc.subcore_barrier()
    obase = cid * NODES_PAD + sid * SLICE
    pltpu.sync_copy(acc_sh.at[pl.ds(sid * SLICE, CE)], pfv0)
    pltpu.sync_copy(pfv0, acc_o.at[pl.ds(obase, CE)])
    pltpu.sync_copy(acc_sh.at[pl.ds(sid * SLICE + CE, SLICE - CE)],
                    pfv0.at[pl.ds(0, SLICE - CE)])
    pltpu.sync_copy(pfv0.at[pl.ds(0, SLICE - CE)],
                    acc_o.at[pl.ds(obase + CE, SLICE - CE)])


_k1 = functools.partial(
    pl.kernel,
    mesh=plsc.VectorSubcoreMesh(core_axis_name="c", subcore_axis_name="s",
                                num_cores=2),
    out_type=[jax.ShapeDtypeStruct((2 * NODES_PAD,), jnp.float32)],
    scratch_types=(
        ([pltpu.VMEM((CE,), jnp.float32)] * 3
         + [pltpu.VMEM((CE,), jnp.int32)] * 2) * NSETS
        + [pltpu.VMEM_SHARED((NODES_PAD,), jnp.float32)]
        + [pltpu.SemaphoreType.DMA] * NSETS
        + [pltpu.SemaphoreType.DMA] * NSETS
    ),
)(_k1_body)


def _k2_body(acc_ref, nd_ref, part_ref, out_ref):
    delta = acc_ref[0] + acc_ref[1]
    rnd = nd_ref[...] * DEM_SCALE + DEM_SHIFT
    cons = jnp.sum(jnp.abs(delta - rnd))
    vcr = jnp.sum(part_ref[:, 0, :])
    flw = jnp.sum(part_ref[:, 1, :])
    total = (W_VCR * vcr / N_EDGES
             + W_FLOW * flw / N_EDGES
             + W_CONS * cons / N_NODES)
    out_ref[...] = jnp.reshape(total, (1, 1))


_k2 = pl.pallas_call(
    _k2_body,
    out_shape=jax.ShapeDtypeStruct((1, 1), jnp.float32),
)


def kernel(pred, y, edge_capacity, edge_flow, net_demand, edge_index):
    eix = edge_index.reshape(2 * N_EDGES)
    pf, part = _k0(pred, y, edge_capacity, edge_flow)
    (acc,) = _k1(pf, eix)
    nd2 = jnp.pad(net_demand, (0, NODES_PAD - N_NODES)).reshape(NODES_PAD // LANE, LANE)
    out = _k2(acc.reshape(2, NODES_PAD // LANE, LANE), nd2, part)
    return out[0, 0]


# speedup vs baseline: 1.0963x; 1.0963x over previous
import functools

import jax
import jax.numpy as jnp
from jax import lax
from jax.experimental import pallas as pl
from jax.experimental.pallas import tpu as pltpu
from jax.experimental.pallas import tpu_sc as plsc

N_NODES = 100000
N_EDGES = 3200000

W_VCR = 1.0
W_FLOW = 0.005
W_CONS = 0.05

TGT_SCALE, TGT_SHIFT = 0.8, 0.5
CAP_SCALE, CAP_SHIFT = 1500.0, 2000.0
FLOW_SCALE, FLOW_SHIFT = 1200.0, 0.0
DEM_SCALE, DEM_SHIFT = 500.0, 0.0

NW = 32
LANE = 128
CR = 40
CE = CR * LANE
NCHUNKS = N_EDGES // CE
CHUNK_LO = NCHUNKS // NW
CHUNK_EXTRA = NCHUNKS - CHUNK_LO * NW
NBLK = CHUNK_LO + 1
NODES_PAD = 100096
SLICE = NODES_PAD // 16
NSETS = 3


def _k1_body(pred_h, y_h, cap_h, flow_h, eix_h, acc_o, part_o, *scr):
    cid = lax.axis_index("c")
    sid = lax.axis_index("s")
    w = cid * 16 + sid
    stage, acc_sh = scr[8 * NSETS], scr[8 * NSETS + 1]
    isems = scr[8 * NSETS + 2:8 * NSETS + 2 + NSETS]
    ssems = scr[8 * NSETS + 2 + NSETS:]
    bufs = [tuple(scr[8 * s:8 * s + 8]) + (isems[s], ssems[s])
            for s in range(NSETS)]
    pfv0 = bufs[0][4]

    def zstep(j, carry):
        for u in range(LANE // 16):
            pfv0[pl.ds(j * LANE + u * 16, 16)] = jnp.zeros((16,), jnp.float32)
        return carry

    lax.fori_loop(0, CR, zstep, 0)
    pltpu.sync_copy(pfv0, acc_sh.at[pl.ds(sid * SLICE, CE)])
    pltpu.sync_copy(pfv0.at[pl.ds(0, SLICE - CE)],
                    acc_sh.at[pl.ds(sid * SLICE + CE, SLICE - CE)])
    plsc.subcore_barrier()

    wmin = jnp.minimum(w, CHUNK_EXTRA)
    base_chunk = CHUNK_LO * w + wmin
    n_chunks = CHUNK_LO + jnp.where(w < CHUNK_EXTRA, 1, 0)

    def prefetch(g, s):
        pv, yv, cv, fv, _, _, sv, dv, isem, _ = bufs[s]
        e0 = g * CE
        pltpu.async_copy(pred_h.at[pl.ds(e0, CE)], pv, isem)
        pltpu.async_copy(y_h.at[pl.ds(e0, CE)], yv, isem)
        pltpu.async_copy(cap_h.at[pl.ds(e0, CE)], cv, isem)
        pltpu.async_copy(flow_h.at[pl.ds(e0, CE)], fv, isem)
        pltpu.async_copy(eix_h.at[pl.ds(e0, CE)], sv, isem)
        pltpu.async_copy(eix_h.at[pl.ds(N_EDGES + e0, CE)], dv, isem)

    def wait_in(s):
        pv, yv, cv, fv, _, _, sv, dv, isem, _ = bufs[s]
        for dst in (pv, yv, cv, fv, sv, dv):
            pltpu.make_async_copy(pred_h.at[pl.ds(0, CE)], dst, isem).wait()

    def compute(s, mvec, carry):
        pv, yv, cv, fv, pfv, npfv, _, _, _, _ = bufs[s]

        def estep(j, c2):
            for u in range(LANE // 16):
                av, af = c2
                o = j * LANE + u * 16
                p = pv[pl.ds(o, 16)]
                yy = yv[pl.ds(o, 16)]
                cc = cv[pl.ds(o, 16)]
                ff = fv[pl.ds(o, 16)]
                pf = (p * TGT_SCALE + TGT_SHIFT) * (cc * CAP_SCALE + CAP_SHIFT)
                d1 = p - yy
                d2 = pf - (ff * FLOW_SCALE + FLOW_SHIFT)
                if mvec is not None:
                    pf = pf * mvec
                    d1 = d1 * mvec
                    d2 = d2 * mvec
                pfv[pl.ds(o, 16)] = pf
                npfv[pl.ds(o, 16)] = -pf
                c2 = (av + d1 * d1, af + d2 * d2)
            return c2

        return lax.fori_loop(0, CR, estep, carry)

    def fire(s):
        _, _, _, _, pfv, npfv, sv, dv, _, ssem = bufs[s]
        pltpu.async_copy(pfv, acc_sh.at[dv], ssem, add=True)
        pltpu.async_copy(npfv, acc_sh.at[sv], ssem, add=True)

    def drain(s):
        _, _, _, _, pfv, npfv, sv, dv, _, ssem = bufs[s]
        pltpu.make_async_copy(pfv, acc_sh.at[dv], ssem).wait()
        pltpu.make_async_copy(npfv, acc_sh.at[sv], ssem).wait()

    zero16 = jnp.zeros((16,), jnp.float32)
    for s in range(NSETS):
        prefetch(base_chunk + s, s)

    wait_in(0)
    carry = compute(0, None, (zero16, zero16))
    fire(0)
    wait_in(1)
    carry = compute(1, None, carry)
    fire(1)

    def body(k, carry):
        for off in (2, 3, 4):
            s = off % NSETS
            c = 3 * k + off
            wait_in(s)
            sd = (s + 1) % NSETS
            drain(sd)
            prefetch(jnp.minimum(base_chunk + c + 1, NCHUNKS - 1), sd)
            if off == 4:
                mvec = jnp.zeros((16,), jnp.float32) + jnp.where(
                    c < n_chunks, 1.0, 0.0).astype(jnp.float32)
            else:
                mvec = None
            carry = compute(s, mvec, carry)
            fire(s)
        return carry

    avcr, aflow = lax.fori_loop(0, (NBLK - 2) // 3, body, carry)
    drain(0)
    drain(1)
    wait_in(2)

    stage[...] = avcr
    pltpu.sync_copy(stage, part_o.at[pl.ds(w * 16, 16)])
    stage[...] = aflow
    pltpu.sync_copy(stage, part_o.at[pl.ds((NW + w) * 16, 16)])

    plsc.subcore_barrier()
    obase = cid * NODES_PAD + sid * SLICE
    pltpu.sync_copy(acc_sh.at[pl.ds(sid * SLICE, CE)], pfv0)
    pltpu.sync_copy(pfv0, acc_o.at[pl.ds(obase, CE)])
    pltpu.sync_copy(acc_sh.at[pl.ds(sid * SLICE + CE, SLICE - CE)],
                    pfv0.at[pl.ds(0, SLICE - CE)])
    pltpu.sync_copy(pfv0.at[pl.ds(0, SLICE - CE)],
                    acc_o.at[pl.ds(obase + CE, SLICE - CE)])


_k1 = functools.partial(
    pl.kernel,
    mesh=plsc.VectorSubcoreMesh(core_axis_name="c", subcore_axis_name="s",
                                num_cores=2),
    out_type=[
        jax.ShapeDtypeStruct((2 * NODES_PAD,), jnp.float32),
        jax.ShapeDtypeStruct((2 * NW * 16,), jnp.float32),
    ],
    scratch_types=(
        ([pltpu.VMEM((CE,), jnp.float32)] * 6
         + [pltpu.VMEM((CE,), jnp.int32)] * 2) * NSETS
        + [
            pltpu.VMEM((16,), jnp.float32),
            pltpu.VMEM_SHARED((NODES_PAD,), jnp.float32),
        ]
        + [pltpu.SemaphoreType.DMA] * NSETS
        + [pltpu.SemaphoreType.DMA] * NSETS
    ),
)(_k1_body)


def _k2_body(acc_ref, nd_ref, part_ref, out_ref):
    delta = acc_ref[0] + acc_ref[1]
    rnd = nd_ref[...] * DEM_SCALE + DEM_SHIFT
    cons = jnp.sum(jnp.abs(delta - rnd))
    vcr = jnp.sum(part_ref[0:NW, :])
    flw = jnp.sum(part_ref[NW:2 * NW, :])
    total = (W_VCR * vcr / N_EDGES
             + W_FLOW * flw / N_EDGES
             + W_CONS * cons / N_NODES)
    out_ref[...] = jnp.reshape(total, (1, 1))


_k2 = pl.pallas_call(
    _k2_body,
    out_shape=jax.ShapeDtypeStruct((1, 1), jnp.float32),
)


def kernel(pred, y, edge_capacity, edge_flow, net_demand, edge_index):
    eix = edge_index.reshape(2 * N_EDGES)
    acc, part = _k1(pred, y, edge_capacity, edge_flow, eix)
    nd2 = jnp.pad(net_demand, (0, NODES_PAD - N_NODES)).reshape(NODES_PAD // LANE, LANE)
    out = _k2(acc.reshape(2, NODES_PAD // LANE, LANE), nd2,
              part.reshape(2 * NW, 16))
    return out[0, 0]
